# baseline (speedup 1.0000x reference)
import functools

import jax
import jax.numpy as jnp
import numpy as np
from jax.experimental import pallas as pl

_CHAIN = 512


def _gcn_body(x_ref, a_ref, w1_ref, b1_ref, w2_ref, b2_ref, o_ref):
    slope = jnp.float32(0.01)
    a = a_ref[...].astype(jnp.bfloat16)
    w1 = w1_ref[...]
    w2 = w2_ref[...]
    b1 = b1_ref[...]
    b2 = b2_ref[...]
    n_chains = x_ref.shape[1] // _CHAIN
    sls = [slice(c * _CHAIN, (c + 1) * _CHAIN) for c in range(n_chains)]
    ss = [jnp.dot(a, x_ref[:, sl], preferred_element_type=jnp.float32)
          for sl in sls]
    hs = []
    for s in ss:
        h = jnp.dot(s.astype(jnp.bfloat16), w1,
                    preferred_element_type=jnp.float32) + b1
        hs.append(jnp.maximum(h, slope * h).astype(jnp.bfloat16))
    s2s = [jnp.dot(a, h, preferred_element_type=jnp.float32) for h in hs]
    for sl, s2 in zip(sls, s2s):
        o = jnp.dot(s2.astype(jnp.bfloat16), w2,
                    preferred_element_type=jnp.float32) + b2
        o_ref[:, sl] = jnp.maximum(o, slope * o)


@functools.partial(jax.jit, static_argnames=("steps",))
def _gcn_block(Xr, A, W1e, b1t, W2e, b2t, steps):
    N = A.shape[0]
    K = Xr.shape[1]
    kin_blk = K // steps
    kout_blk = kin_blk * W2e.shape[1] // W1e.shape[0]
    return pl.pallas_call(
        _gcn_body,
        grid=(steps,),
        in_specs=[
            pl.BlockSpec((N, kin_blk), lambda g: (0, g)),
            pl.BlockSpec((N, N), lambda g: (0, 0)),
            pl.BlockSpec((W1e.shape[0], W1e.shape[1]), lambda g: (0, 0)),
            pl.BlockSpec((1, W1e.shape[1]), lambda g: (0, 0)),
            pl.BlockSpec((W2e.shape[0], W2e.shape[1]), lambda g: (0, 0)),
            pl.BlockSpec((1, W2e.shape[1]), lambda g: (0, 0)),
        ],
        out_specs=pl.BlockSpec((N, kout_blk), lambda g: (0, g)),
        out_shape=jax.ShapeDtypeStruct((N, steps * kout_blk), jnp.float32),
    )(Xr, A, W1e, b1t, W2e, b2t)


def kernel(X, A, W1, b1, W2, b2):
    B, N, T, F_in = X.shape
    F_sp = W1.shape[1]
    BT = B * T
    grp = _CHAIN // F_in
    steps = 2

    Xr = jnp.transpose(X, (1, 0, 2, 3)).reshape(N, BT * F_in).astype(jnp.bfloat16)

    mask1 = np.kron(np.eye(grp, dtype=np.float32),
                    np.ones((F_in, F_sp), np.float32))
    mask2 = np.kron(np.eye(grp, dtype=np.float32),
                    np.ones((F_sp, F_sp), np.float32))
    W1e = (jnp.tile(W1, (grp, grp)) * mask1).astype(jnp.bfloat16)
    W2e = (jnp.tile(W2, (grp, grp)) * mask2).astype(jnp.bfloat16)
    b1t = jnp.tile(b1, grp)[None, :]
    b2t = jnp.tile(b2, grp)[None, :]

    out = _gcn_block(Xr, A, W1e, b1t, W2e, b2t, steps)
    return out.reshape(N, B, T, F_sp).transpose(1, 0, 2, 3)


# speedup vs baseline: 1.1536x; 1.1536x over previous
import functools

import jax
import jax.numpy as jnp
import numpy as np
from jax.experimental import pallas as pl

_CHAIN = 128


def _gcn_body(x_ref, a_ref, w1_ref, b1_ref, w2_ref, b2_ref, o_ref):
    slope = jnp.float32(0.01)
    a = a_ref[...].astype(jnp.bfloat16)
    w1 = w1_ref[...]
    w2 = w2_ref[...]
    b1 = b1_ref[...]
    b2 = b2_ref[...]
    n_chains = x_ref.shape[1] // _CHAIN
    sls = [slice(c * _CHAIN, (c + 1) * _CHAIN) for c in range(n_chains)]
    ss = [jnp.dot(a, x_ref[:, sl], preferred_element_type=jnp.float32)
          for sl in sls]
    hs = []
    for s in ss:
        h = jnp.dot(s.astype(jnp.bfloat16), w1,
                    preferred_element_type=jnp.float32) + b1
        hs.append(jnp.maximum(h, slope * h).astype(jnp.bfloat16))
    s2s = [jnp.dot(a, h, preferred_element_type=jnp.float32) for h in hs]
    for sl, s2 in zip(sls, s2s):
        o = jnp.dot(s2.astype(jnp.bfloat16), w2,
                    preferred_element_type=jnp.float32) + b2
        o_ref[:, sl] = jnp.maximum(o, slope * o)


@functools.partial(jax.jit, static_argnames=("steps",))
def _gcn_block(Xr, A, W1e, b1t, W2e, b2t, steps):
    N = A.shape[0]
    K = Xr.shape[1]
    kin_blk = K // steps
    kout_blk = kin_blk * W2e.shape[1] // W1e.shape[0]
    return pl.pallas_call(
        _gcn_body,
        grid=(steps,),
        in_specs=[
            pl.BlockSpec((N, kin_blk), lambda g: (0, g)),
            pl.BlockSpec((N, N), lambda g: (0, 0)),
            pl.BlockSpec((W1e.shape[0], W1e.shape[1]), lambda g: (0, 0)),
            pl.BlockSpec((1, W1e.shape[1]), lambda g: (0, 0)),
            pl.BlockSpec((W2e.shape[0], W2e.shape[1]), lambda g: (0, 0)),
            pl.BlockSpec((1, W2e.shape[1]), lambda g: (0, 0)),
        ],
        out_specs=pl.BlockSpec((N, kout_blk), lambda g: (0, g)),
        out_shape=jax.ShapeDtypeStruct((N, steps * kout_blk), jnp.float32),
    )(Xr, A, W1e, b1t, W2e, b2t)


def kernel(X, A, W1, b1, W2, b2):
    B, N, T, F_in = X.shape
    F_sp = W1.shape[1]
    BT = B * T
    grp = _CHAIN // F_in
    steps = 2

    Xr = jnp.transpose(X, (1, 0, 2, 3)).reshape(N, BT * F_in).astype(jnp.bfloat16)

    mask1 = np.kron(np.eye(grp, dtype=np.float32),
                    np.ones((F_in, F_sp), np.float32))
    mask2 = np.kron(np.eye(grp, dtype=np.float32),
                    np.ones((F_sp, F_sp), np.float32))
    W1e = (jnp.tile(W1, (grp, grp)) * mask1).astype(jnp.bfloat16)
    W2e = (jnp.tile(W2, (grp, grp)) * mask2).astype(jnp.bfloat16)
    b1t = jnp.tile(b1, grp)[None, :]
    b2t = jnp.tile(b2, grp)[None, :]

    out = _gcn_block(Xr, A, W1e, b1t, W2e, b2t, steps)
    return out.reshape(N, B, T, F_sp).transpose(1, 0, 2, 3)


# speedup vs baseline: 1.4207x; 1.2315x over previous
import functools

import jax
import jax.numpy as jnp
import numpy as np
from jax.experimental import pallas as pl

_CHAIN = 256


def _gcn_body(x_ref, a_ref, w1_ref, b1_ref, w2_ref, b2_ref, o_ref):
    slope = jnp.float32(0.01)
    a = a_ref[...].astype(jnp.bfloat16)
    w1 = w1_ref[...]
    w2 = w2_ref[...]
    b1 = b1_ref[...]
    b2 = b2_ref[...]
    n_chains = x_ref.shape[1] // _CHAIN
    sls = [slice(c * _CHAIN, (c + 1) * _CHAIN) for c in range(n_chains)]
    ss = [jnp.dot(a, x_ref[:, sl], preferred_element_type=jnp.float32)
          for sl in sls]
    hs = []
    for s in ss:
        h = jnp.dot(s.astype(jnp.bfloat16), w1,
                    preferred_element_type=jnp.float32) + b1
        hs.append(jnp.maximum(h, slope * h).astype(jnp.bfloat16))
    s2s = [jnp.dot(a, h, preferred_element_type=jnp.float32) for h in hs]
    for sl, s2 in zip(sls, s2s):
        o = jnp.dot(s2.astype(jnp.bfloat16), w2,
                    preferred_element_type=jnp.float32) + b2
        o_ref[:, sl] = jnp.maximum(o, slope * o).astype(jnp.bfloat16)


@functools.partial(jax.jit, static_argnames=("steps",))
def _gcn_block(Xr, A, W1e, b1t, W2e, b2t, steps):
    N = A.shape[0]
    K = Xr.shape[1]
    kin_blk = K // steps
    kout_blk = kin_blk * W2e.shape[1] // W1e.shape[0]
    return pl.pallas_call(
        _gcn_body,
        grid=(steps,),
        in_specs=[
            pl.BlockSpec((N, kin_blk), lambda g: (0, g)),
            pl.BlockSpec((N, N), lambda g: (0, 0)),
            pl.BlockSpec((W1e.shape[0], W1e.shape[1]), lambda g: (0, 0)),
            pl.BlockSpec((1, W1e.shape[1]), lambda g: (0, 0)),
            pl.BlockSpec((W2e.shape[0], W2e.shape[1]), lambda g: (0, 0)),
            pl.BlockSpec((1, W2e.shape[1]), lambda g: (0, 0)),
        ],
        out_specs=pl.BlockSpec((N, kout_blk), lambda g: (0, g)),
        out_shape=jax.ShapeDtypeStruct((N, steps * kout_blk), jnp.bfloat16),
    )(Xr, A, W1e, b1t, W2e, b2t)


def kernel(X, A, W1, b1, W2, b2):
    B, N, T, F_in = X.shape
    F_sp = W1.shape[1]
    BT = B * T
    grp = _CHAIN // F_in
    steps = 2

    Xr = jnp.transpose(X, (1, 0, 2, 3)).reshape(N, BT * F_in).astype(jnp.bfloat16)

    mask1 = np.kron(np.eye(grp, dtype=np.float32),
                    np.ones((F_in, F_sp), np.float32))
    mask2 = np.kron(np.eye(grp, dtype=np.float32),
                    np.ones((F_sp, F_sp), np.float32))
    W1e = (jnp.tile(W1, (grp, grp)) * mask1).astype(jnp.bfloat16)
    W2e = (jnp.tile(W2, (grp, grp)) * mask2).astype(jnp.bfloat16)
    b1t = jnp.tile(b1, grp)[None, :]
    b2t = jnp.tile(b2, grp)[None, :]

    out = _gcn_block(Xr, A, W1e, b1t, W2e, b2t, steps)
    return out.reshape(N, B, T, F_sp).transpose(1, 0, 2, 3).astype(jnp.float32)
